# baseline (speedup 1.0000x reference)
import jax
import jax.numpy as jnp
from jax.experimental import pallas as pl
from jax.experimental.pallas import tpu as pltpu

_OUT = 299
_GBLK = 16


def _upsample_body(f_ref, x_ref, o_ref):
    si = pl.program_id(0)
    gi = pl.program_id(1)

    i2 = jax.lax.broadcasted_iota(jnp.int32, (_OUT, _OUT), 0).astype(jnp.float32)
    j2 = jax.lax.broadcasted_iota(jnp.int32, (_OUT, _OUT), 1).astype(jnp.float32)

    for gsub in range(_GBLK):
        g = gi * _GBLK + gsub
        tlx = f_ref[si, g, 0]
        tly = f_ref[si, g, 1]
        brx = f_ref[si, g, 2]
        bry = f_ref[si, g, 3]
        hc = (brx - tlx).astype(jnp.float32)
        wc = (bry - tly).astype(jnp.float32)

        src_r = jnp.clip((i2 + 0.5) * (hc / _OUT) - 0.5, 0.0, hc - 1.0) \
            + tlx.astype(jnp.float32)
        rmat = jnp.maximum(0.0, 1.0 - jnp.abs(j2 - src_r))

        src_c = jnp.clip((j2 + 0.5) * (wc / _OUT) - 0.5, 0.0, wc - 1.0) \
            + tly.astype(jnp.float32)
        cmat = jnp.maximum(0.0, 1.0 - jnp.abs(i2 - src_c))

        for ch in range(3):
            mid = jnp.dot(x_ref[0, ch], cmat, preferred_element_type=jnp.float32)
            o_ref[0, ch, :, gsub, :] = jnp.dot(rmat, mid,
                                               preferred_element_type=jnp.float32)


def kernel(x, f):
    s, g = f.shape[0], f.shape[1]
    grid_spec = pltpu.PrefetchScalarGridSpec(
        num_scalar_prefetch=1,
        grid=(s, g // _GBLK),
        in_specs=[
            pl.BlockSpec((1, 3, _OUT, _OUT),
                         lambda si, gi, fp: (si, 0, 0, 0)),
        ],
        out_specs=pl.BlockSpec((1, 3, _OUT, _GBLK, _OUT),
                               lambda si, gi, fp: (si, 0, 0, gi, 0)),
    )
    out5 = pl.pallas_call(
        _upsample_body,
        grid_spec=grid_spec,
        out_shape=jax.ShapeDtypeStruct((s, 3, _OUT, g, _OUT), jnp.float32),
        compiler_params=pltpu.CompilerParams(
            dimension_semantics=("parallel", "arbitrary"),
            vmem_limit_bytes=56 * 1024 * 1024,
        ),
        name="roi_bilinear_upsample",
    )(f, x)
    return jnp.transpose(out5, (0, 3, 1, 2, 4))


# speedup vs baseline: 1.0471x; 1.0471x over previous
import jax
import jax.numpy as jnp
from jax.experimental import pallas as pl
from jax.experimental.pallas import tpu as pltpu

_OUT = 299
_GBLK = 8


def _upsample_body(f_ref, x_ref, o_ref):
    si = pl.program_id(0)
    gi = pl.program_id(1)

    i2 = jax.lax.broadcasted_iota(jnp.int32, (_OUT, _OUT), 0).astype(jnp.float32)
    j2 = jax.lax.broadcasted_iota(jnp.int32, (_OUT, _OUT), 1).astype(jnp.float32)

    for gsub in range(_GBLK):
        g = gi * _GBLK + gsub
        tlx = f_ref[si, g, 0]
        tly = f_ref[si, g, 1]
        brx = f_ref[si, g, 2]
        bry = f_ref[si, g, 3]
        hc = (brx - tlx).astype(jnp.float32)
        wc = (bry - tly).astype(jnp.float32)

        src_r = jnp.clip((i2 + 0.5) * (hc / _OUT) - 0.5, 0.0, hc - 1.0) \
            + tlx.astype(jnp.float32)
        rmat = jnp.maximum(0.0, 1.0 - jnp.abs(j2 - src_r))

        src_c = jnp.clip((j2 + 0.5) * (wc / _OUT) - 0.5, 0.0, wc - 1.0) \
            + tly.astype(jnp.float32)
        cmat = jnp.maximum(0.0, 1.0 - jnp.abs(i2 - src_c))

        for ch in range(3):
            mid = jnp.dot(x_ref[0, ch], cmat, preferred_element_type=jnp.float32)
            o_ref[0, ch, :, gsub, :] = jnp.dot(rmat, mid,
                                               preferred_element_type=jnp.float32)


def kernel(x, f):
    s, g = f.shape[0], f.shape[1]
    grid_spec = pltpu.PrefetchScalarGridSpec(
        num_scalar_prefetch=1,
        grid=(s, g // _GBLK),
        in_specs=[
            pl.BlockSpec((1, 3, _OUT, _OUT),
                         lambda si, gi, fp: (si, 0, 0, 0)),
        ],
        out_specs=pl.BlockSpec((1, 3, _OUT, _GBLK, _OUT),
                               lambda si, gi, fp: (si, 0, 0, gi, 0)),
    )
    out5 = pl.pallas_call(
        _upsample_body,
        grid_spec=grid_spec,
        out_shape=jax.ShapeDtypeStruct((s, 3, _OUT, g, _OUT), jnp.float32),
        compiler_params=pltpu.CompilerParams(
            dimension_semantics=("parallel", "arbitrary"),
        ),
        name="roi_bilinear_upsample",
    )(f, x)
    return jnp.transpose(out5, (0, 3, 1, 2, 4))
